# Initial kernel scaffold; baseline (speedup 1.0000x reference)
import jax
import jax.numpy as jnp
from jax.experimental import pallas as pl


def kernel(features, class_ids, stages, shared_protos):
    raise NotImplementedError("write your pallas kernel here")


# speedup vs baseline: 1.3284x; 1.3284x over previous
import functools

import jax
import jax.numpy as jnp
from jax import lax
from jax.experimental import pallas as pl
from jax.experimental.pallas import tpu as pltpu
from jax.experimental.pallas import tpu_sc as plsc

B = 16384
D = 128
S = 8
P = 16
L = 16
NC = 2
NS = 16
NW = NC * NS
RPW = B // NW
CHUNK = 128
NCHUNK = RPW // CHUNK


def _body(feat_hbm, stages_hbm, protos_hbm, out_hbm, means_hbm,
          protos_v, means_v, idx_raw, idx2, feat_v, rows_v, sem_f, sem_g):
    wid = lax.axis_index("s") * NC + lax.axis_index("c")
    base = wid * RPW

    pltpu.sync_copy(stages_hbm.at[pl.ds(base, RPW)], idx_raw)

    pltpu.sync_copy(protos_hbm, protos_v)
    for s in range(S):
        for j in range(D // L):
            sl = pl.ds(j * L, L)
            acc = protos_v[s, 0, sl]
            for p in range(1, P):
                acc = acc + protos_v[s, p, sl]
            means_v[s, sl] = acc * (1.0 / (2 * P))

    pltpu.sync_copy(means_v, means_hbm.at[pl.ds(wid * S, S)])
    off = wid * S
    for c in range(NCHUNK):
        for j in range(CHUNK // L):
            idx2[c, pl.ds(j * L, L)] = idx_raw[pl.ds(c * CHUNK + j * L, L)] + off

    for c in range(NCHUNK):
        r0 = base + c * CHUNK
        cp_f = pltpu.async_copy(feat_hbm.at[pl.ds(r0, CHUNK)], feat_v, sem_f)
        cp_g = pltpu.async_copy(means_hbm.at[idx2.at[c]], rows_v, sem_g)
        cp_f.wait()
        cp_g.wait()

        def row_body(r, carry):
            for j in range(D // L):
                sl = pl.ds(j * L, L)
                feat_v[r, sl] = feat_v[r, sl] + rows_v[r, sl]
            return carry

        lax.fori_loop(0, CHUNK, row_body, 0)
        pltpu.sync_copy(feat_v, out_hbm.at[pl.ds(r0, CHUNK)])


_sc_call = functools.partial(
    pl.kernel,
    out_type=(
        jax.ShapeDtypeStruct((B, D), jnp.float32),
        jax.ShapeDtypeStruct((NW * S, D), jnp.float32),
    ),
    mesh=plsc.VectorSubcoreMesh(core_axis_name="c", subcore_axis_name="s"),
    scratch_types=[
        pltpu.VMEM((S, P, D), jnp.float32),
        pltpu.VMEM((S, D), jnp.float32),
        pltpu.VMEM((RPW,), jnp.int32),
        pltpu.VMEM((NCHUNK, CHUNK), jnp.int32),
        pltpu.VMEM((CHUNK, D), jnp.float32),
        pltpu.VMEM((CHUNK, D), jnp.float32),
        pltpu.SemaphoreType.DMA,
        pltpu.SemaphoreType.DMA,
    ],
)(_body)


def kernel(features, class_ids, stages, shared_protos):
    del class_ids
    out, _ = _sc_call(features, stages.astype(jnp.int32), shared_protos)
    return out
